# baseline (speedup 1.0000x reference)
import functools
import math

import jax
import jax.numpy as jnp
from jax import lax
from jax.experimental import pallas as pl
from jax.experimental.pallas import tpu as pltpu
from jax.experimental.pallas import tpu_sc as plsc

_BINS = 64
_EDGES = 65
_EPAD = 80
_HW = 384 * 384
_PLANES = 6
_NT = 2 * _PLANES
_NW = 32
_SLICE = _HW // _NW
_VREGS = _SLICE // 16
_ECHUNK = 13
_CEDGE = [math.exp(0.78125 * j) for j in range(_EDGES)]


def _sc_hist_kernel(x_hbm, out_hbm, xbuf, vbuf, accbuf):
    c = lax.axis_index("c")
    s = lax.axis_index("s")
    wid = s * 2 + c

    zero16 = jnp.zeros((16,), jnp.float32)
    for pt in range(_NT):
        pltpu.sync_copy(x_hbm.at[pt, pl.ds(wid * _SLICE, _SLICE)], xbuf)

        def expbody(i, carry):
            xv = xbuf[pl.ds(i * 16, 16)]
            vbuf[pl.ds(i * 16, 16)] = jnp.exp(xv * -50.0)
            return carry

        lax.fori_loop(0, _VREGS, expbody, 0)

        for chunk in range(5):
            base = chunk * _ECHUNK

            def body(i, accs):
                v = vbuf[pl.ds(i * 16, 16)]
                out = []
                for k in range(_ECHUNK):
                    q = v * _CEDGE[base + k]
                    out.append(accs[k] + 1.0 / (1.0 + q * q))
                return tuple(out)

            accs = lax.fori_loop(0, _VREGS, body,
                                 tuple(zero16 for _ in range(_ECHUNK)))
            for k in range(_ECHUNK):
                accbuf[pl.ds((pt * _EPAD + base + k) * 16, 16)] = accs[k]
        for j in range(_EDGES, _EPAD):
            accbuf[pl.ds((pt * _EPAD + j) * 16, 16)] = zero16

    pltpu.sync_copy(accbuf, out_hbm.at[wid])


_sc_hist = functools.partial(
    pl.kernel,
    out_type=jax.ShapeDtypeStruct((_NW, _NT * _EPAD * 16), jnp.float32),
    mesh=plsc.VectorSubcoreMesh(core_axis_name="c", subcore_axis_name="s"),
    scratch_types=[
        pltpu.VMEM((_SLICE,), jnp.float32),
        pltpu.VMEM((_SLICE,), jnp.float32),
        pltpu.VMEM((_NT * _EPAD * 16,), jnp.float32),
    ],
)(_sc_hist_kernel)


def _loss_kernel(acc_ref, loss_ref):
    total = jnp.zeros((1, 1), jnp.float32)
    for p in range(_PLANES):
        t_o = jnp.sum(acc_ref[p], axis=1, keepdims=True)
        t_t = jnp.sum(acc_ref[p + _PLANES], axis=1, keepdims=True)
        d_o = t_o[0:_BINS] - t_o[1:_EDGES]
        d_t = t_t[0:_BINS] - t_t[1:_EDGES]
        total = total + jnp.full((1, 1), jnp.sum(jnp.abs(d_o - d_t)))
    loss_ref[...] = total * (1.0 / (_PLANES * _BINS * _HW))


@jax.jit
def kernel(output, target):
    o = output.reshape(_PLANES, _HW)
    t = target.reshape(_PLANES, _HW)
    x = jnp.concatenate([o, t], axis=0)
    parts = _sc_hist(x).reshape(_NW, _NT, _EPAD, 16)
    acc = jnp.transpose(parts, (1, 2, 0, 3)).reshape(_NT, _EPAD, _NW * 16)
    loss = pl.pallas_call(
        _loss_kernel,
        out_shape=jax.ShapeDtypeStruct((1, 1), jnp.float32),
    )(acc)
    return loss[0, 0]


# speedup vs baseline: 2.6380x; 2.6380x over previous
import functools
import math

import jax
import jax.numpy as jnp
from jax import lax
from jax.experimental import pallas as pl
from jax.experimental.pallas import tpu as pltpu
from jax.experimental.pallas import tpu_sc as plsc

_BINS = 64
_EDGES = 65
_HW = 384 * 384
_LANES = 128
_ROWS = _HW // _LANES
_PLANES = 6
_NT = 2 * _PLANES
_HALF_SD = 100.0 / (2 * _BINS)

_NSC = 2
_NTC = _NT - _NSC

_EPAD_TC = 72
_UNROLL = 8


def _tc_acc_kernel(x_ref, acc_ref):
    dvec = _HALF_SD * lax.broadcasted_iota(jnp.int32, (_EPAD_TC, 1), 0).astype(jnp.float32)

    def body(k, accs):
        tile = x_ref[0, pl.ds(k * _UNROLL, _UNROLL), :] * 50.0
        new = []
        for u in range(_UNROLL):
            row = tile[u:u + 1, :]
            t = jnp.tanh(jnp.broadcast_to(row, (_EPAD_TC, _LANES)) - dvec)
            new.append(accs[u % 2] + t if u < 2 else new[u - 2] + t)
        return (new[_UNROLL - 2], new[_UNROLL - 1])

    zero = jnp.zeros((_EPAD_TC, _LANES), jnp.float32)
    accs = lax.fori_loop(0, _ROWS // _UNROLL, body, (zero, zero))
    acc_ref[0] = accs[0] + accs[1]


_NW = 32
_SLICE = _HW // _NW
_VREGS = _SLICE // 16
_ECHUNK = 13
_EPAD_SC = 80
_CEDGE = [math.exp(0.78125 * j) for j in range(_EDGES)]


def _sc_hist_kernel(x_hbm, out_hbm, xbuf, vbuf, accbuf):
    c = lax.axis_index("c")
    s = lax.axis_index("s")
    wid = s * 2 + c

    zero16 = jnp.zeros((16,), jnp.float32)
    for pt in range(_NSC):
        pltpu.sync_copy(x_hbm.at[pt, pl.ds(wid * _SLICE, _SLICE)], xbuf)

        def expbody(i, carry):
            xv = xbuf[pl.ds(i * 16, 16)]
            vbuf[pl.ds(i * 16, 16)] = jnp.exp(xv * -50.0)
            return carry

        lax.fori_loop(0, _VREGS, expbody, 0)

        for chunk in range(5):
            base = chunk * _ECHUNK

            def body(i, accs):
                v = vbuf[pl.ds(i * 16, 16)]
                out = []
                for k in range(_ECHUNK):
                    q = v * _CEDGE[base + k]
                    out.append(accs[k] + 1.0 / (1.0 + q * q))
                return tuple(out)

            accs = lax.fori_loop(0, _VREGS, body,
                                 tuple(zero16 for _ in range(_ECHUNK)))
            for k in range(_ECHUNK):
                accbuf[pl.ds((pt * _EPAD_SC + base + k) * 16, 16)] = accs[k]
        for j in range(_EDGES, _EPAD_SC):
            accbuf[pl.ds((pt * _EPAD_SC + j) * 16, 16)] = zero16

    pltpu.sync_copy(accbuf, out_hbm.at[wid])


_sc_hist = functools.partial(
    pl.kernel,
    out_type=jax.ShapeDtypeStruct((_NW, _NSC * _EPAD_SC * 16), jnp.float32),
    mesh=plsc.VectorSubcoreMesh(core_axis_name="c", subcore_axis_name="s"),
    scratch_types=[
        pltpu.VMEM((_SLICE,), jnp.float32),
        pltpu.VMEM((_SLICE,), jnp.float32),
        pltpu.VMEM((_NSC * _EPAD_SC * 16,), jnp.float32),
    ],
)(_sc_hist_kernel)


def _loss_kernel(sc_ref, tc_ref, loss_ref):
    def sc_col(i):
        blk = sc_ref[pl.ds(i * _EPAD_SC, _EPAD_SC), :]
        return jnp.sum(blk, axis=1, keepdims=True)

    def tc_col(i):
        blk = tc_ref[pl.ds(i * _EPAD_TC, _EPAD_TC), :]
        return 0.5 * jnp.sum(blk, axis=1, keepdims=True)

    total = jnp.zeros((1, 1), jnp.float32)
    for p in range(_PLANES):
        t_o = sc_col(p) if p < _NSC else tc_col(p - _NSC)
        t_t = tc_col(_PLANES - _NSC + p)
        d_o = t_o[0:_BINS] - t_o[1:_EDGES]
        d_t = t_t[0:_BINS] - t_t[1:_EDGES]
        total = total + jnp.full((1, 1), jnp.sum(jnp.abs(d_o - d_t)))
    loss_ref[...] = total * (1.0 / (_PLANES * _BINS * _HW))


@jax.jit
def kernel(output, target):
    o = output.reshape(_PLANES, _ROWS, _LANES)
    t = target.reshape(_PLANES, _ROWS, _LANES)
    x_sc = output.reshape(_PLANES, _HW)[:_NSC]
    x_tc = jnp.concatenate([o[_NSC:], t], axis=0)

    parts = _sc_hist(x_sc).reshape(_NW, _NSC, _EPAD_SC, 16)
    acc_sc = jnp.transpose(parts, (1, 2, 0, 3)).reshape(_NSC, _EPAD_SC, _NW * 16)

    acc_tc = pl.pallas_call(
        _tc_acc_kernel,
        grid=(_NTC,),
        in_specs=[pl.BlockSpec((1, _ROWS, _LANES), lambda p: (p, 0, 0))],
        out_specs=pl.BlockSpec((1, _EPAD_TC, _LANES), lambda p: (p, 0, 0)),
        out_shape=jax.ShapeDtypeStruct((_NTC, _EPAD_TC, _LANES), jnp.float32),
    )(x_tc)

    loss = pl.pallas_call(
        _loss_kernel,
        out_shape=jax.ShapeDtypeStruct((1, 1), jnp.float32),
    )(acc_sc.reshape(_NSC * _EPAD_SC, _NW * 16),
      acc_tc.reshape(_NTC * _EPAD_TC, _LANES))
    return loss[0, 0]


# speedup vs baseline: 2.9444x; 1.1161x over previous
import jax
import jax.numpy as jnp
from jax import lax
from jax.experimental import pallas as pl

_BINS = 64
_EDGES = 65
_EPAD = 72
_LANES = 128
_HW = 384 * 384
_ROWS = _HW // _LANES
_PLANES = 6
_HALF_SD = 100.0 / (2 * _BINS)
_UNROLL = 8


def _plane_kernel(o_ref, t_ref, loss_ref):
    p = pl.program_id(0)
    dvec = _HALF_SD * lax.broadcasted_iota(jnp.int32, (_EPAD, 1), 0).astype(jnp.float32)

    def sweep(x_ref, sign, accs):
        def body(k, accs):
            tile = x_ref[0, pl.ds(k * _UNROLL, _UNROLL), :] * 50.0
            new = []
            for u in range(_UNROLL):
                row = tile[u:u + 1, :]
                t = jnp.tanh(jnp.broadcast_to(row, (_EPAD, _LANES)) - dvec)
                prev = accs[u % 2] if u < 2 else new[u - 2]
                new.append(prev + t if sign > 0 else prev - t)
            return (new[_UNROLL - 2], new[_UNROLL - 1])

        return lax.fori_loop(0, _ROWS // _UNROLL, body, accs)

    zero = jnp.zeros((_EPAD, _LANES), jnp.float32)
    accs = sweep(o_ref, 1, (zero, zero))
    accs = sweep(t_ref, -1, accs)
    td = jnp.sum(accs[0] + accs[1], axis=1, keepdims=True)
    d = td[0:_BINS] - td[1:_EDGES]
    partial = 0.5 * jnp.sum(jnp.abs(d))

    @pl.when(p == 0)
    def _():
        loss_ref[...] = jnp.zeros((1, 1), jnp.float32)

    loss_ref[...] += jnp.full((1, 1), partial)

    @pl.when(p == _PLANES - 1)
    def _():
        loss_ref[...] = loss_ref[...] * (1.0 / (_PLANES * _BINS * _HW))


@jax.jit
def kernel(output, target):
    o = output.reshape(_PLANES, _ROWS, _LANES)
    t = target.reshape(_PLANES, _ROWS, _LANES)
    loss = pl.pallas_call(
        _plane_kernel,
        grid=(_PLANES,),
        in_specs=[
            pl.BlockSpec((1, _ROWS, _LANES), lambda p: (p, 0, 0)),
            pl.BlockSpec((1, _ROWS, _LANES), lambda p: (p, 0, 0)),
        ],
        out_specs=pl.BlockSpec((1, 1), lambda p: (0, 0)),
        out_shape=jax.ShapeDtypeStruct((1, 1), jnp.float32),
    )(o, t)
    return loss[0, 0]


# speedup vs baseline: 3.3242x; 1.1290x over previous
import jax
import jax.numpy as jnp
from jax import lax
from jax.experimental import pallas as pl

_BINS = 64
_EDGES = 65
_EPAD = 72
_LANES = 128
_HW = 384 * 384
_ROWS = _HW // _LANES
_PLANES = 6
_HALF_SD = 100.0 / (2 * _BINS)
_UNROLL = 16


def _plane_kernel(o_ref, t_ref, loss_ref):
    p = pl.program_id(0)
    dvec = _HALF_SD * lax.broadcasted_iota(jnp.int32, (_EPAD, 1), 0).astype(jnp.float32)

    def sweep(x_ref, sign, accs):
        def body(k, accs):
            tile = x_ref[0, pl.ds(k * _UNROLL, _UNROLL), :] * 50.0
            new = []
            for u in range(_UNROLL):
                row = tile[u:u + 1, :]
                t = jnp.tanh(jnp.broadcast_to(row, (_EPAD, _LANES)) - dvec)
                prev = accs[u % 2] if u < 2 else new[u - 2]
                new.append(prev + t if sign > 0 else prev - t)
            return (new[_UNROLL - 2], new[_UNROLL - 1])

        return lax.fori_loop(0, _ROWS // _UNROLL, body, accs)

    zero = jnp.zeros((_EPAD, _LANES), jnp.float32)
    accs = sweep(o_ref, 1, (zero, zero))
    accs = sweep(t_ref, -1, accs)
    td = jnp.sum(accs[0] + accs[1], axis=1, keepdims=True)
    d = td[0:_BINS] - td[1:_EDGES]
    partial = 0.5 * jnp.sum(jnp.abs(d))

    @pl.when(p == 0)
    def _():
        loss_ref[...] = jnp.zeros((1, 1), jnp.float32)

    loss_ref[...] += jnp.full((1, 1), partial)

    @pl.when(p == _PLANES - 1)
    def _():
        loss_ref[...] = loss_ref[...] * (1.0 / (_PLANES * _BINS * _HW))


@jax.jit
def kernel(output, target):
    o = output.reshape(_PLANES, _ROWS, _LANES)
    t = target.reshape(_PLANES, _ROWS, _LANES)
    loss = pl.pallas_call(
        _plane_kernel,
        grid=(_PLANES,),
        in_specs=[
            pl.BlockSpec((1, _ROWS, _LANES), lambda p: (p, 0, 0)),
            pl.BlockSpec((1, _ROWS, _LANES), lambda p: (p, 0, 0)),
        ],
        out_specs=pl.BlockSpec((1, 1), lambda p: (0, 0)),
        out_shape=jax.ShapeDtypeStruct((1, 1), jnp.float32),
    )(o, t)
    return loss[0, 0]


# speedup vs baseline: 3.4641x; 1.0421x over previous
import jax
import jax.numpy as jnp
from jax import lax
from jax.experimental import pallas as pl

_BINS = 64
_EDGES = 65
_EPAD = 72
_LANES = 128
_HW = 384 * 384
_ROWS = _HW // _LANES
_PLANES = 6
_HALF_SD = 100.0 / (2 * _BINS)
_UNROLL = 32


def _plane_kernel(o_ref, t_ref, loss_ref):
    p = pl.program_id(0)
    dvec = _HALF_SD * lax.broadcasted_iota(jnp.int32, (_EPAD, 1), 0).astype(jnp.float32)

    def sweep(x_ref, sign, accs):
        def body(k, accs):
            tile = x_ref[0, pl.ds(k * _UNROLL, _UNROLL), :] * 50.0
            new = []
            for u in range(_UNROLL):
                row = tile[u:u + 1, :]
                t = jnp.tanh(jnp.broadcast_to(row, (_EPAD, _LANES)) - dvec)
                prev = accs[u % 2] if u < 2 else new[u - 2]
                new.append(prev + t if sign > 0 else prev - t)
            return (new[_UNROLL - 2], new[_UNROLL - 1])

        return lax.fori_loop(0, _ROWS // _UNROLL, body, accs)

    zero = jnp.zeros((_EPAD, _LANES), jnp.float32)
    accs = sweep(o_ref, 1, (zero, zero))
    accs = sweep(t_ref, -1, accs)
    td = jnp.sum(accs[0] + accs[1], axis=1, keepdims=True)
    d = td[0:_BINS] - td[1:_EDGES]
    partial = 0.5 * jnp.sum(jnp.abs(d))

    @pl.when(p == 0)
    def _():
        loss_ref[...] = jnp.zeros((1, 1), jnp.float32)

    loss_ref[...] += jnp.full((1, 1), partial)

    @pl.when(p == _PLANES - 1)
    def _():
        loss_ref[...] = loss_ref[...] * (1.0 / (_PLANES * _BINS * _HW))


@jax.jit
def kernel(output, target):
    o = output.reshape(_PLANES, _ROWS, _LANES)
    t = target.reshape(_PLANES, _ROWS, _LANES)
    loss = pl.pallas_call(
        _plane_kernel,
        grid=(_PLANES,),
        in_specs=[
            pl.BlockSpec((1, _ROWS, _LANES), lambda p: (p, 0, 0)),
            pl.BlockSpec((1, _ROWS, _LANES), lambda p: (p, 0, 0)),
        ],
        out_specs=pl.BlockSpec((1, 1), lambda p: (0, 0)),
        out_shape=jax.ShapeDtypeStruct((1, 1), jnp.float32),
    )(o, t)
    return loss[0, 0]


# speedup vs baseline: 3.5425x; 1.0226x over previous
import jax
import jax.numpy as jnp
from jax import lax
from jax.experimental import pallas as pl

_BINS = 64
_EDGES = 65
_EPAD = 72
_LANES = 128
_HW = 384 * 384
_ROWS = _HW // _LANES
_PLANES = 6
_HALF_SD = 100.0 / (2 * _BINS)
_UNROLL = 48


def _plane_kernel(o_ref, t_ref, loss_ref):
    p = pl.program_id(0)
    dvec = _HALF_SD * lax.broadcasted_iota(jnp.int32, (_EPAD, 1), 0).astype(jnp.float32)

    def sweep(x_ref, sign, accs):
        def body(k, accs):
            tile = x_ref[0, pl.ds(k * _UNROLL, _UNROLL), :] * 50.0
            new = []
            for u in range(_UNROLL):
                row = tile[u:u + 1, :]
                t = jnp.tanh(jnp.broadcast_to(row, (_EPAD, _LANES)) - dvec)
                prev = accs[u % 2] if u < 2 else new[u - 2]
                new.append(prev + t if sign > 0 else prev - t)
            return (new[_UNROLL - 2], new[_UNROLL - 1])

        return lax.fori_loop(0, _ROWS // _UNROLL, body, accs)

    zero = jnp.zeros((_EPAD, _LANES), jnp.float32)
    accs = sweep(o_ref, 1, (zero, zero))
    accs = sweep(t_ref, -1, accs)
    td = jnp.sum(accs[0] + accs[1], axis=1, keepdims=True)
    d = td[0:_BINS] - td[1:_EDGES]
    partial = 0.5 * jnp.sum(jnp.abs(d))

    @pl.when(p == 0)
    def _():
        loss_ref[...] = jnp.zeros((1, 1), jnp.float32)

    loss_ref[...] += jnp.full((1, 1), partial)

    @pl.when(p == _PLANES - 1)
    def _():
        loss_ref[...] = loss_ref[...] * (1.0 / (_PLANES * _BINS * _HW))


@jax.jit
def kernel(output, target):
    o = output.reshape(_PLANES, _ROWS, _LANES)
    t = target.reshape(_PLANES, _ROWS, _LANES)
    loss = pl.pallas_call(
        _plane_kernel,
        grid=(_PLANES,),
        in_specs=[
            pl.BlockSpec((1, _ROWS, _LANES), lambda p: (p, 0, 0)),
            pl.BlockSpec((1, _ROWS, _LANES), lambda p: (p, 0, 0)),
        ],
        out_specs=pl.BlockSpec((1, 1), lambda p: (0, 0)),
        out_shape=jax.ShapeDtypeStruct((1, 1), jnp.float32),
    )(o, t)
    return loss[0, 0]
